# baseline (device time: 246305 ns/iter reference)
import jax
import jax.numpy as jnp
from jax import lax
from jax.experimental import pallas as pl
from jax.experimental.pallas import tpu as pltpu

N_DEV = 32


def kernel(x, w_mat):
    m_per, k = x.shape
    _, n_per = w_mat.shape

    def body(x_ref, w_ref, out_ref, comm_ref, send_sems, recv_sems):
        my = lax.axis_index("i")
        left = lax.rem(my - 1 + N_DEV, N_DEV)
        right = lax.rem(my + 1, N_DEV)

        barrier = pltpu.get_barrier_semaphore()
        for nbr in (left, right):
            pl.semaphore_signal(
                barrier, inc=1,
                device_id=(nbr,), device_id_type=pl.DeviceIdType.MESH,
            )
        pl.semaphore_wait(barrier, 2)

        def gemm(origin, chunk):
            y = jnp.dot(chunk, w_ref[:, :], preferred_element_type=jnp.float32)
            y = y * jax.nn.sigmoid(y)
            out_ref[pl.ds(origin * m_per, m_per), :] = y

        comm_ref[0, :, :] = x_ref[:, :]
        gemm(my, x_ref[:, :])

        for h in range(N_DEV - 1):
            rdma = pltpu.make_async_remote_copy(
                src_ref=comm_ref.at[h],
                dst_ref=comm_ref.at[h + 1],
                send_sem=send_sems.at[h],
                recv_sem=recv_sems.at[h + 1],
                device_id=(right,),
                device_id_type=pl.DeviceIdType.MESH,
            )
            rdma.start()
            rdma.wait()
            origin = lax.rem(my - h - 1 + N_DEV, N_DEV)
            gemm(origin, comm_ref[h + 1, :, :])

    return pl.pallas_call(
        body,
        out_shape=jax.ShapeDtypeStruct((k, n_per), jnp.float32),
        in_specs=[
            pl.BlockSpec(memory_space=pltpu.VMEM),
            pl.BlockSpec(memory_space=pltpu.VMEM),
        ],
        out_specs=pl.BlockSpec(memory_space=pltpu.VMEM),
        scratch_shapes=[
            pltpu.VMEM((N_DEV, m_per, k), jnp.float32),
            pltpu.SemaphoreType.DMA((N_DEV,)),
            pltpu.SemaphoreType.DMA((N_DEV,)),
        ],
        compiler_params=pltpu.CompilerParams(collective_id=0),
    )(x, w_mat)


# device time: 187396 ns/iter; 1.3144x vs baseline; 1.3144x over previous
import jax
import jax.numpy as jnp
from jax import lax
from jax.experimental import pallas as pl
from jax.experimental.pallas import tpu as pltpu

N_DEV = 32
R_HOPS = 16
L_HOPS = 15


def kernel(x, w_mat):
    m_per, k = x.shape
    _, n_per = w_mat.shape

    def body(x_ref, w_ref, out_ref, comm_ref, send_sems, recv_sems):
        my = lax.axis_index("i")
        left = lax.rem(my - 1 + N_DEV, N_DEV)
        right = lax.rem(my + 1, N_DEV)

        barrier = pltpu.get_barrier_semaphore()
        for nbr in (left, right):
            pl.semaphore_signal(
                barrier, inc=1,
                device_id=(nbr,), device_id_type=pl.DeviceIdType.MESH,
            )
        pl.semaphore_wait(barrier, 2)

        def send_r(step):
            return pltpu.make_async_remote_copy(
                src_ref=comm_ref.at[step],
                dst_ref=comm_ref.at[step + 1],
                send_sem=send_sems.at[step + 1],
                recv_sem=recv_sems.at[step + 1],
                device_id=(right,),
                device_id_type=pl.DeviceIdType.MESH,
            )

        def send_l(step):
            src = 0 if step == 0 else 16 + step
            return pltpu.make_async_remote_copy(
                src_ref=comm_ref.at[src],
                dst_ref=comm_ref.at[17 + step],
                send_sem=send_sems.at[17 + step],
                recv_sem=recv_sems.at[17 + step],
                device_id=(left,),
                device_id_type=pl.DeviceIdType.MESH,
            )

        def gemm(origin, chunk):
            y = jnp.dot(chunk, w_ref[:, :], preferred_element_type=jnp.float32)
            y = y * jax.nn.sigmoid(y)
            out_ref[pl.ds(origin * m_per, m_per), :] = y

        comm_ref[0, :, :] = x_ref[:, :]
        send_r(0).start()
        send_l(0).start()
        gemm(my, x_ref[:, :])

        for step in range(1, R_HOPS + 1):
            send_r(step - 1).wait_recv()
            if step < R_HOPS:
                send_r(step).start()
            if step <= L_HOPS:
                send_l(step - 1).wait_recv()
                if step < L_HOPS:
                    send_l(step).start()
            gemm(lax.rem(my - step + N_DEV, N_DEV), comm_ref[step, :, :])
            if step <= L_HOPS:
                gemm(lax.rem(my + step, N_DEV), comm_ref[16 + step, :, :])

        for step in range(R_HOPS):
            send_r(step).wait_send()
        for step in range(L_HOPS):
            send_l(step).wait_send()

    return pl.pallas_call(
        body,
        out_shape=jax.ShapeDtypeStruct((k, n_per), jnp.float32),
        in_specs=[
            pl.BlockSpec(memory_space=pltpu.VMEM),
            pl.BlockSpec(memory_space=pltpu.VMEM),
        ],
        out_specs=pl.BlockSpec(memory_space=pltpu.VMEM),
        scratch_shapes=[
            pltpu.VMEM((N_DEV, m_per, k), jnp.float32),
            pltpu.SemaphoreType.DMA((N_DEV,)),
            pltpu.SemaphoreType.DMA((N_DEV,)),
        ],
        compiler_params=pltpu.CompilerParams(collective_id=0),
    )(x, w_mat)


# device time: 130144 ns/iter; 1.8926x vs baseline; 1.4399x over previous
import jax
import jax.numpy as jnp
from jax import lax
from jax.experimental import pallas as pl
from jax.experimental.pallas import tpu as pltpu

N_DEV = 32
R_HOPS = 16
L_HOPS = 15

def _ham_cycle_coords():
    path = []
    for z in range(4):
        ys = range(4) if z % 2 == 0 else range(3, -1, -1)
        path += [(0, y, z) for y in ys]
    for z in range(3, -1, -1):
        ys = range(4) if z % 2 == 1 else range(3, -1, -1)
        path += [(1, y, z) for y in ys]
    return path


_ROW_OFF = {(0, 0): 0, (1, 0): 1, (1, 1): 2, (0, 1): 3,
            (0, 2): 4, (1, 2): 5, (1, 3): 6, (0, 3): 7}


def _mesh_idx(c):
    x, y, z = c
    return z * 8 + _ROW_OFF[(x, y)]


_PATH = _ham_cycle_coords()
RING = [_mesh_idx(c) for c in _PATH]
INV = [0] * N_DEV
for _pos, _m in enumerate(RING):
    INV[_m] = _pos


def kernel(x, w_mat):
    m_per, k = x.shape
    _, n_per = w_mat.shape

    my = lax.axis_index("i")
    ring = jnp.array(RING, jnp.int32)
    inv = jnp.array(INV, jnp.int32)
    p = inv[my]
    left = ring[(p - 1) % N_DEV]
    right = ring[(p + 1) % N_DEV]
    origin_by_slot = jnp.concatenate([
        my[None].astype(jnp.int32),
        ring[(p - jnp.arange(1, R_HOPS + 1)) % N_DEV],
        ring[(p + jnp.arange(1, L_HOPS + 1)) % N_DEV],
    ])
    meta = jnp.concatenate(
        [left[None].astype(jnp.int32), right[None].astype(jnp.int32),
         origin_by_slot]
    )

    def body(x_ref, w_ref, meta_ref, out_ref, comm_ref, send_sems, recv_sems):
        lft = meta_ref[0]
        rgt = meta_ref[1]

        barrier = pltpu.get_barrier_semaphore()
        for nbr in (lft, rgt):
            pl.semaphore_signal(
                barrier, inc=1,
                device_id=(nbr,), device_id_type=pl.DeviceIdType.MESH,
            )
        pl.semaphore_wait(barrier, 2)

        def send_r(step):
            return pltpu.make_async_remote_copy(
                src_ref=comm_ref.at[step],
                dst_ref=comm_ref.at[step + 1],
                send_sem=send_sems.at[step + 1],
                recv_sem=recv_sems.at[step + 1],
                device_id=(rgt,),
                device_id_type=pl.DeviceIdType.MESH,
            )

        def send_l(step):
            src = 0 if step == 0 else 16 + step
            return pltpu.make_async_remote_copy(
                src_ref=comm_ref.at[src],
                dst_ref=comm_ref.at[17 + step],
                send_sem=send_sems.at[17 + step],
                recv_sem=recv_sems.at[17 + step],
                device_id=(lft,),
                device_id_type=pl.DeviceIdType.MESH,
            )

        def gemm(slot, chunk):
            origin = meta_ref[2 + slot]
            y = jnp.dot(chunk, w_ref[:, :], preferred_element_type=jnp.float32)
            y = y * jax.nn.sigmoid(y)
            out_ref[pl.ds(origin * m_per, m_per), :] = y

        comm_ref[0, :, :] = x_ref[:, :]
        send_r(0).start()
        send_l(0).start()
        gemm(0, x_ref[:, :])

        for step in range(1, R_HOPS + 1):
            send_r(step - 1).wait_recv()
            if step < R_HOPS:
                send_r(step).start()
            if step <= L_HOPS:
                send_l(step - 1).wait_recv()
                if step < L_HOPS:
                    send_l(step).start()
            gemm(step, comm_ref[step, :, :])
            if step <= L_HOPS:
                gemm(16 + step, comm_ref[16 + step, :, :])

        for step in range(R_HOPS):
            send_r(step).wait_send()
        for step in range(L_HOPS):
            send_l(step).wait_send()

    return pl.pallas_call(
        body,
        out_shape=jax.ShapeDtypeStruct((k, n_per), jnp.float32),
        in_specs=[
            pl.BlockSpec(memory_space=pltpu.VMEM),
            pl.BlockSpec(memory_space=pltpu.VMEM),
            pl.BlockSpec(memory_space=pltpu.SMEM),
        ],
        out_specs=pl.BlockSpec(memory_space=pltpu.VMEM),
        scratch_shapes=[
            pltpu.VMEM((N_DEV, m_per, k), jnp.float32),
            pltpu.SemaphoreType.DMA((N_DEV,)),
            pltpu.SemaphoreType.DMA((N_DEV,)),
        ],
        compiler_params=pltpu.CompilerParams(collective_id=0),
    )(x, w_mat, meta)


# device time: 104053 ns/iter; 2.3671x vs baseline; 1.2507x over previous
import jax
import jax.numpy as jnp
from jax import lax
from jax.experimental import pallas as pl
from jax.experimental.pallas import tpu as pltpu

N_DEV = 32
R_HOPS = 16
L_HOPS = 15
N_SEG = 2

def _ham_cycle_coords():
    path = []
    for z in range(4):
        ys = range(4) if z % 2 == 0 else range(3, -1, -1)
        path += [(0, y, z) for y in ys]
    for z in range(3, -1, -1):
        ys = range(4) if z % 2 == 1 else range(3, -1, -1)
        path += [(1, y, z) for y in ys]
    return path


_ROW_OFF = {(0, 0): 0, (1, 0): 1, (1, 1): 2, (0, 1): 3,
            (0, 2): 4, (1, 2): 5, (1, 3): 6, (0, 3): 7}


def _mesh_idx(c):
    x, y, z = c
    return z * 8 + _ROW_OFF[(x, y)]


RING = [_mesh_idx(c) for c in _ham_cycle_coords()]
INV = [0] * N_DEV
for _pos, _m in enumerate(RING):
    INV[_m] = _pos


def kernel(x, w_mat):
    m_per, k = x.shape
    _, n_per = w_mat.shape
    m_seg = m_per // N_SEG

    my = lax.axis_index("i")
    ring = jnp.array(RING, jnp.int32)
    inv = jnp.array(INV, jnp.int32)
    p = inv[my]
    left = ring[(p - 1) % N_DEV]
    right = ring[(p + 1) % N_DEV]
    meta = jnp.concatenate([
        left[None].astype(jnp.int32),
        right[None].astype(jnp.int32),
        my[None].astype(jnp.int32),
        ring[(p - jnp.arange(1, R_HOPS + 1)) % N_DEV],
        ring[(p + jnp.arange(1, L_HOPS + 1)) % N_DEV],
    ])

    def body(x_ref, w_ref, meta_ref, out_ref, comm_ref, send_sems, recv_sems):
        lft = meta_ref[0]
        rgt = meta_ref[1]

        barrier = pltpu.get_barrier_semaphore()
        for nbr in (lft, rgt):
            pl.semaphore_signal(
                barrier, inc=1,
                device_id=(nbr,), device_id_type=pl.DeviceIdType.MESH,
            )
        pl.semaphore_wait(barrier, 2)

        def hop(step, direction, seg):
            src_slot, src_dir = (0, 0) if step == 0 else (step, direction)
            rows = pl.ds(seg * m_seg, m_seg)
            return pltpu.make_async_remote_copy(
                src_ref=comm_ref.at[src_slot, src_dir, rows],
                dst_ref=comm_ref.at[step + 1, direction, rows],
                send_sem=send_sems.at[step + 1, direction, seg],
                recv_sem=recv_sems.at[step + 1, direction, seg],
                device_id=(rgt if direction == 0 else lft,),
                device_id_type=pl.DeviceIdType.MESH,
            )

        def silu_store(y, origin):
            y = y * jax.nn.sigmoid(y)
            out_ref[pl.ds(origin * m_per, m_per), :] = y

        comm_ref[0, 0, :, :] = x_ref[:, :]
        for seg in range(N_SEG):
            hop(0, 0, seg).start()
            hop(0, 1, seg).start()

        y0 = jnp.dot(x_ref[:, :], w_ref[:, :], preferred_element_type=jnp.float32)
        silu_store(y0, meta_ref[2])

        for step in range(1, R_HOPS + 1):
            has_l = step <= L_HOPS
            for seg in range(N_SEG):
                hop(step - 1, 0, seg).wait_recv()
                if step < R_HOPS:
                    hop(step, 0, seg).start()
                if has_l:
                    hop(step - 1, 1, seg).wait_recv()
                    if step < L_HOPS:
                        hop(step, 1, seg).start()
            if has_l:
                pair = jnp.reshape(comm_ref[step, :, :, :], (2 * m_per, k))
                y = jnp.dot(pair, w_ref[:, :], preferred_element_type=jnp.float32)
                silu_store(y[:m_per, :], meta_ref[2 + step])
                silu_store(y[m_per:, :], meta_ref[2 + R_HOPS + step])
            else:
                y = jnp.dot(comm_ref[step, 0, :, :], w_ref[:, :],
                            preferred_element_type=jnp.float32)
                silu_store(y, meta_ref[2 + step])

        for step in range(R_HOPS):
            for seg in range(N_SEG):
                hop(step, 0, seg).wait_send()
                if step < L_HOPS:
                    hop(step, 1, seg).wait_send()

    return pl.pallas_call(
        body,
        out_shape=jax.ShapeDtypeStruct((k, n_per), jnp.float32),
        in_specs=[
            pl.BlockSpec(memory_space=pltpu.VMEM),
            pl.BlockSpec(memory_space=pltpu.VMEM),
            pl.BlockSpec(memory_space=pltpu.SMEM),
        ],
        out_specs=pl.BlockSpec(memory_space=pltpu.VMEM),
        scratch_shapes=[
            pltpu.VMEM((R_HOPS + 1, 2, m_per, k), jnp.float32),
            pltpu.SemaphoreType.DMA((R_HOPS + 1, 2, N_SEG)),
            pltpu.SemaphoreType.DMA((R_HOPS + 1, 2, N_SEG)),
        ],
        compiler_params=pltpu.CompilerParams(collective_id=0),
    )(x, w_mat, meta)
